# Initial kernel scaffold; baseline (speedup 1.0000x reference)
import jax
import jax.numpy as jnp
from jax.experimental import pallas as pl


def kernel(x):
    raise NotImplementedError("write your pallas kernel here")


# speedup vs baseline: 9.2972x; 9.2972x over previous
import jax
import jax.numpy as jnp
from jax.experimental import pallas as pl

_K = 26214
_ROWS = 128
_COLS = 32768
_BLK_ROWS = 8


def _monotonic_key(x):
    b = jax.lax.bitcast_convert_type(x, jnp.uint32)
    neg = b >= jnp.uint32(0x80000000)
    return jnp.where(neg, ~b, b | jnp.uint32(0x80000000))


def _body(x_ref, o_ref):
    x = x_ref[...]
    ukey = _monotonic_key(x)

    def step(i, prefix):
        bit = jnp.uint32(1) << (jnp.uint32(31) - i.astype(jnp.uint32))
        cand = prefix | bit
        cnt = jnp.sum((ukey >= cand).astype(jnp.int32), axis=1, keepdims=True)
        return jnp.where(cnt >= _K, cand, prefix)

    prefix = jnp.zeros((_BLK_ROWS, 1), jnp.uint32)
    thresh = jax.lax.fori_loop(0, 32, step, prefix)
    o_ref[...] = jnp.where(ukey >= thresh, x, jnp.float32(0.0))


def kernel(x):
    grid = _ROWS // _BLK_ROWS
    return pl.pallas_call(
        _body,
        grid=(grid,),
        in_specs=[pl.BlockSpec((_BLK_ROWS, _COLS), lambda i: (i, 0))],
        out_specs=pl.BlockSpec((_BLK_ROWS, _COLS), lambda i: (i, 0)),
        out_shape=jax.ShapeDtypeStruct((_ROWS, _COLS), jnp.float32),
    )(x)
